# baseline (speedup 1.0000x reference)
import functools

import jax
import jax.numpy as jnp
from jax import lax
from jax.experimental import pallas as pl
from jax.experimental.pallas import tpu as pltpu
from jax.experimental.pallas import tpu_sc as plsc

N_P, N_D, N_SE = 10000, 3000, 994
E = 320000
IN_DIM, HID, OUT = 512, 128, 994

NC, NS = 2, 16
NW = NC * NS
CH = 128
EPW = 10240
E_PAD = NW * EPW
NWIN = EPW // CH

P_PAD, D_PAD, SE_PAD = 10240, 3072, 1024
CW = 8


@functools.lru_cache(maxsize=None)
def _make_seg_sum(n_dst_pad: int, with_counts: bool):
    NSLOT = 2 if n_dst_pad >= P_PAD else 4
    rows_per_sub = n_dst_pad // NS
    mesh = plsc.VectorSubcoreMesh(core_axis_name="c", subcore_axis_name="s",
                                  num_cores=NC, num_subcores=NS)

    out_type = [jax.ShapeDtypeStruct((NC, n_dst_pad, HID), jnp.float32)]
    if with_counts:
        out_type.append(jax.ShapeDtypeStruct((NC * n_dst_pad,), jnp.float32))

    DS = 2 * NSLOT
    scratch = (
        [pltpu.VMEM((CH,), jnp.int32) for _ in range(NSLOT)]
        + [pltpu.VMEM((CH,), jnp.int32) for _ in range(DS)]
        + [pltpu.VMEM((CH, HID), jnp.float32) for _ in range(NSLOT)]
        + [pltpu.SemaphoreType.DMA for _ in range(NSLOT)]
        + [pltpu.SemaphoreType.DMA for _ in range(NSLOT)]
        + [pltpu.SemaphoreType.DMA for _ in range(NSLOT)]
        + [pltpu.VMEM_SHARED((n_dst_pad, HID), jnp.float32)]
    )
    if with_counts:
        scratch += [pltpu.VMEM((CH,), jnp.float32),
                    pltpu.VMEM((rows_per_sub,), jnp.float32),
                    pltpu.VMEM_SHARED((n_dst_pad,), jnp.float32)]

    def body(src_hbm, dst_hbm, table_hbm, zrow_hbm, *out_and_scratch):
        if with_counts:
            out_hbm, cnt_hbm = out_and_scratch[:2]
            rest = list(out_and_scratch[2:])
        else:
            out_hbm = out_and_scratch[0]
            rest = list(out_and_scratch[1:])
        isrc = [rest.pop(0) for _ in range(NSLOT)]
        idst = [rest.pop(0) for _ in range(DS)]
        rows = [rest.pop(0) for _ in range(NSLOT)]
        gsem = [rest.pop(0) for _ in range(NSLOT)]
        isem = [rest.pop(0) for _ in range(NSLOT)]
        ssem = [rest.pop(0) for _ in range(NSLOT)]
        acc = rest.pop(0)
        if with_counts:
            ones_v = rest.pop(0)
            cbuf = rest.pop(0)
            cacc = rest.pop(0)

        c = lax.axis_index("c")
        s = lax.axis_index("s")
        wid = c * NS + s
        ebase = wid * EPW
        r0 = s * rows_per_sub

        pltpu.sync_copy(zrow_hbm.at[pl.ds(r0, rows_per_sub)],
                        acc.at[pl.ds(r0, rows_per_sub)])
        if with_counts:
            for i in range(rows_per_sub // 16):
                cbuf[pl.ds(i * 16, 16)] = jnp.zeros((16,), jnp.float32)
            for i in range(CH // 16):
                ones_v[pl.ds(i * 16, 16)] = jnp.ones((16,), jnp.float32)
            pltpu.sync_copy(cbuf, cacc.at[pl.ds(r0, rows_per_sub)])
        plsc.subcore_barrier()

        def idx_start(t, u, w):
            off = ebase + w * CH
            pltpu.async_copy(src_hbm.at[pl.ds(off, CH)], isrc[t], isem[t])
            pltpu.async_copy(dst_hbm.at[pl.ds(off, CH)], idst[u], isem[t])

        def scatter_wait(t, u_prev):
            pltpu.make_async_copy(rows[t], acc.at[idst[u_prev]],
                                  ssem[t]).wait()
            if with_counts:
                pltpu.make_async_copy(ones_v, cacc.at[idst[u_prev]],
                                      ssem[t]).wait()

        def gather_start(t, u, w, may_have_prev_scatter):
            off = ebase + w * CH
            pltpu.make_async_copy(src_hbm.at[pl.ds(off, CH)], isrc[t],
                                  isem[t]).wait()
            pltpu.make_async_copy(dst_hbm.at[pl.ds(off, CH)], idst[u],
                                  isem[t]).wait()
            if may_have_prev_scatter:
                @pl.when(w >= NSLOT)
                def _():
                    scatter_wait(t, (u - NSLOT) % DS)
            pltpu.async_copy(table_hbm.at[isrc[t]], rows[t], gsem[t])

        for t in range(NSLOT):
            idx_start(t, t, t)
        for t in range(NSLOT - 1):
            gather_start(t, t, t, may_have_prev_scatter=False)

        def grp(k, _):
            for u in range(DS):
                w = k * DS + u
                t = u % NSLOT

                @pl.when(w < NWIN)
                def _drain():
                    pltpu.make_async_copy(table_hbm.at[isrc[t]], rows[t],
                                          gsem[t]).wait()
                    pltpu.async_copy(rows[t], acc.at[idst[u]], ssem[t],
                                     add=True)
                    if with_counts:
                        pltpu.async_copy(ones_v, cacc.at[idst[u]], ssem[t],
                                         add=True)

                @pl.when(w + NSLOT < NWIN)
                def _prefetch():
                    idx_start(t, (u + NSLOT) % DS, w + NSLOT)

                v = w + NSLOT - 1
                uv = (u + NSLOT - 1) % DS
                tv = uv % NSLOT

                @pl.when(v < NWIN)
                def _gather():
                    gather_start(tv, uv, v, may_have_prev_scatter=True)
            return ()

        lax.fori_loop(0, pl.cdiv(NWIN, DS), grp, ())

        for w in range(NWIN - NSLOT, NWIN):
            scatter_wait(w % NSLOT, w % DS)

        plsc.subcore_barrier()
        pltpu.sync_copy(acc.at[pl.ds(r0, rows_per_sub)],
                        out_hbm.at[c, pl.ds(r0, rows_per_sub)])
        if with_counts:
            pltpu.sync_copy(cacc.at[pl.ds(r0, rows_per_sub)], cbuf)
            pltpu.sync_copy(cbuf,
                            cnt_hbm.at[pl.ds(c * n_dst_pad + r0, rows_per_sub)])

    return pl.kernel(body, out_type=tuple(out_type), mesh=mesh,
                     scratch_types=tuple(scratch))


def _seg_sum(src_idx, dst_idx, table, n_dst_pad, with_counts):
    n_dst = {P_PAD: N_P, D_PAD: N_D, SE_PAD: N_SE}[n_dst_pad]
    pad_src = jnp.arange(E_PAD - E, dtype=jnp.int32) % table.shape[0]
    pad_dst = n_dst + jnp.arange(E_PAD - E, dtype=jnp.int32) % (n_dst_pad - n_dst)
    src_idx = jnp.concatenate([src_idx, pad_src])
    dst_idx = jnp.concatenate([dst_idx, pad_dst])
    zrow = jnp.zeros((n_dst_pad, HID), jnp.float32)
    k = _make_seg_sum(n_dst_pad, with_counts)
    res = k(src_idx, dst_idx, table, zrow)
    if with_counts:
        sums, cnt_flat = res
        cnt = jnp.broadcast_to(
            cnt_flat.reshape(NC, n_dst_pad, 1), (NC, n_dst_pad, CW))
        return sums, cnt
    return res


def _dotT(a, w):
    return lax.dot_general(a, w, (((1,), (1,)), ((), ())),
                           preferred_element_type=jnp.float32)


def _proj(x, w, b, act, bn):
    n = x.shape[0]

    def body(x_ref, w_ref, b_ref, o_ref):
        h = _dotT(x_ref[...], w_ref[...]) + b_ref[...]
        o_ref[...] = jnp.tanh(h) if act else h

    return pl.pallas_call(
        body,
        grid=(n // bn,),
        in_specs=[pl.BlockSpec((bn, IN_DIM), lambda i: (i, 0)),
                  pl.BlockSpec((HID, IN_DIM), lambda i: (0, 0)),
                  pl.BlockSpec((1, HID), lambda i: (0, 0))],
        out_specs=pl.BlockSpec((bn, HID), lambda i: (i, 0)),
        out_shape=jax.ShapeDtypeStruct((n, HID), jnp.float32),
    )(x, w, b.reshape(1, HID))


def _combine(convs, h_dst, wr_sum, b_sum, bn):
    n = h_dst.shape[0]
    nj = len(convs)

    def body(*refs):
        sums = refs[0:nj]
        cnts = refs[nj:2 * nj]
        wls = refs[2 * nj:3 * nj]
        h_ref, wr_ref, b_ref, o_ref = refs[3 * nj:]
        out = _dotT(h_ref[...], wr_ref[...]) + b_ref[...]
        for j in range(nj):
            ssum = sums[j][0] + sums[j][1]
            cnt = cnts[j][0, :, 0:1] + cnts[j][1, :, 0:1]
            agg = ssum / jnp.maximum(cnt, 1.0)
            out = out + _dotT(agg, wls[j][...])
        o_ref[...] = out

    in_specs = (
        [pl.BlockSpec((2, bn, HID), lambda i: (0, i, 0)) for _ in range(nj)]
        + [pl.BlockSpec((2, bn, CW), lambda i: (0, i, 0)) for _ in range(nj)]
        + [pl.BlockSpec((HID, HID), lambda i: (0, 0)) for _ in range(nj)]
        + [pl.BlockSpec((bn, HID), lambda i: (i, 0)),
           pl.BlockSpec((HID, HID), lambda i: (0, 0)),
           pl.BlockSpec((1, HID), lambda i: (0, 0))]
    )
    args = ([c[0] for c in convs] + [c[1] for c in convs]
            + [c[2] for c in convs] + [h_dst, wr_sum, b_sum.reshape(1, HID)])
    return pl.pallas_call(
        body,
        grid=(n // bn,),
        in_specs=in_specs,
        out_specs=pl.BlockSpec((bn, HID), lambda i: (i, 0)),
        out_shape=jax.ShapeDtypeStruct((n, HID), jnp.float32),
    )(*args)


def _readout(h, assis, w, b, n_rows, bn=512):
    n_out = w.shape[0]

    def body(h_ref, a_ref, w_ref, b_ref, o_ref):
        o_ref[...] = _dotT(h_ref[...] + a_ref[...], w_ref[...]) + b_ref[...]

    return pl.pallas_call(
        body,
        grid=(pl.cdiv(n_rows, bn),),
        in_specs=[pl.BlockSpec((bn, HID), lambda i: (i, 0)),
                  pl.BlockSpec((bn, HID), lambda i: (i, 0)),
                  pl.BlockSpec((n_out, HID), lambda i: (0, 0)),
                  pl.BlockSpec((1, n_out), lambda i: (0, 0))],
        out_specs=pl.BlockSpec((bn, n_out), lambda i: (i, 0)),
        out_shape=jax.ShapeDtypeStruct((n_rows, n_out), jnp.float32),
    )(h, assis, w, b.reshape(1, n_out))


def kernel(x_patient, x_drug, x_SE, ei_p_drug, ei_rev_p_drug, ei_p_se,
           ei_rev_p_se, params):
    p = params
    xse = jnp.pad(x_SE, ((0, SE_PAD - N_SE), (0, 0)))

    h_p = _proj(x_patient, p["W_in"], p["b_in"], act=True, bn=400)
    h_d = _proj(x_drug, p["W_in"], p["b_in"], act=True, bn=600)
    h_se = _proj(xse, p["W_se"], p["b_se"], act=True, bn=512)
    assis = _proj(x_patient, p["W_cl"], p["b_cl"], act=False, bn=400)

    edges = {
        "pd": (ei_p_drug[0], ei_p_drug[1], D_PAD),
        "pse": (ei_p_se[0], ei_p_se[1], SE_PAD),
        "rpd": (ei_rev_p_drug[0], ei_rev_p_drug[1], P_PAD),
        "rpse": (ei_rev_p_se[0], ei_rev_p_se[1], P_PAD),
    }
    tables = {"pd": "p", "pse": "p", "rpd": "d", "rpse": "se"}

    cnts = {}
    for l in range(2):
        sp = p["sage"][l]
        h = {"p": h_p, "d": h_d, "se": h_se}
        sums = {}
        for et, (src, dst, npad) in edges.items():
            res = _seg_sum(src, dst, h[tables[et]], npad, with_counts=(l == 0))
            if l == 0:
                sums[et], cnts[et] = res
            else:
                (sums[et],) = res

        h_d = _combine([(sums["pd"], cnts["pd"], sp["pd"]["Wl"])],
                       h["d"], sp["pd"]["Wr"], sp["pd"]["bl"], bn=600)
        h_se = _combine([(sums["pse"], cnts["pse"], sp["pse"]["Wl"])],
                        h["se"], sp["pse"]["Wr"], sp["pse"]["bl"], bn=512)
        h_p = _combine([(sums["rpd"], cnts["rpd"], sp["rpd"]["Wl"]),
                        (sums["rpse"], cnts["rpse"], sp["rpse"]["Wl"])],
                       h["p"], sp["rpd"]["Wr"] + sp["rpse"]["Wr"],
                       sp["rpd"]["bl"] + sp["rpse"]["bl"], bn=400)

    return _readout(h_p, assis, p["W_ro"], p["b_ro"], n_rows=N_P - 1)


# speedup vs baseline: 1.1167x; 1.1167x over previous
import functools

import jax
import jax.numpy as jnp
from jax import lax
from jax.experimental import pallas as pl
from jax.experimental.pallas import tpu as pltpu
from jax.experimental.pallas import tpu_sc as plsc

N_P, N_D, N_SE = 10000, 3000, 994
E = 320000
IN_DIM, HID, OUT = 512, 128, 994

NC, NS = 2, 16
NW = NC * NS

P_PAD, D_PAD, SE_PAD = 10240, 3072, 1024
CW = 8


def _variant(n_dst_pad):
    ch = 80 if n_dst_pad >= P_PAD else 128
    epw = ch * ((E + NW * ch - 1) // (NW * ch))
    return ch, epw


@functools.lru_cache(maxsize=None)
def _make_seg_sum(n_dst_pad: int, with_counts: bool):
    NSLOT = 4
    CH, EPW = _variant(n_dst_pad)
    NWIN = EPW // CH
    rows_per_sub = n_dst_pad // NS
    mesh = plsc.VectorSubcoreMesh(core_axis_name="c", subcore_axis_name="s",
                                  num_cores=NC, num_subcores=NS)

    out_type = [jax.ShapeDtypeStruct((NC, n_dst_pad, HID), jnp.float32)]
    if with_counts:
        out_type.append(jax.ShapeDtypeStruct((NC * n_dst_pad,), jnp.float32))

    DS = 2 * NSLOT
    scratch = (
        [pltpu.VMEM((CH,), jnp.int32) for _ in range(NSLOT)]
        + [pltpu.VMEM((CH,), jnp.int32) for _ in range(DS)]
        + [pltpu.VMEM((CH, HID), jnp.float32) for _ in range(NSLOT)]
        + [pltpu.SemaphoreType.DMA for _ in range(NSLOT)]
        + [pltpu.SemaphoreType.DMA for _ in range(NSLOT)]
        + [pltpu.SemaphoreType.DMA for _ in range(NSLOT)]
        + [pltpu.VMEM_SHARED((n_dst_pad, HID), jnp.float32)]
    )
    if with_counts:
        scratch += [pltpu.VMEM((CH,), jnp.float32),
                    pltpu.VMEM((rows_per_sub,), jnp.float32),
                    pltpu.VMEM_SHARED((n_dst_pad,), jnp.float32)]

    def body(src_hbm, dst_hbm, table_hbm, zrow_hbm, *out_and_scratch):
        if with_counts:
            out_hbm, cnt_hbm = out_and_scratch[:2]
            rest = list(out_and_scratch[2:])
        else:
            out_hbm = out_and_scratch[0]
            rest = list(out_and_scratch[1:])
        isrc = [rest.pop(0) for _ in range(NSLOT)]
        idst = [rest.pop(0) for _ in range(DS)]
        rows = [rest.pop(0) for _ in range(NSLOT)]
        gsem = [rest.pop(0) for _ in range(NSLOT)]
        isem = [rest.pop(0) for _ in range(NSLOT)]
        ssem = [rest.pop(0) for _ in range(NSLOT)]
        acc = rest.pop(0)
        if with_counts:
            ones_v = rest.pop(0)
            cbuf = rest.pop(0)
            cacc = rest.pop(0)

        c = lax.axis_index("c")
        s = lax.axis_index("s")
        wid = c * NS + s
        ebase = wid * EPW
        r0 = s * rows_per_sub

        pltpu.sync_copy(zrow_hbm.at[pl.ds(r0, rows_per_sub)],
                        acc.at[pl.ds(r0, rows_per_sub)])
        if with_counts:
            for i in range(rows_per_sub // 16):
                cbuf[pl.ds(i * 16, 16)] = jnp.zeros((16,), jnp.float32)
            for i in range(CH // 16):
                ones_v[pl.ds(i * 16, 16)] = jnp.ones((16,), jnp.float32)
            pltpu.sync_copy(cbuf, cacc.at[pl.ds(r0, rows_per_sub)])
        plsc.subcore_barrier()

        def idx_start(t, u, w):
            off = ebase + w * CH
            pltpu.async_copy(src_hbm.at[pl.ds(off, CH)], isrc[t], isem[t])
            pltpu.async_copy(dst_hbm.at[pl.ds(off, CH)], idst[u], isem[t])

        def scatter_wait(t, u_prev):
            pltpu.make_async_copy(rows[t], acc.at[idst[u_prev]],
                                  ssem[t]).wait()
            if with_counts:
                pltpu.make_async_copy(ones_v, cacc.at[idst[u_prev]],
                                      ssem[t]).wait()

        def gather_start(t, u, w, may_have_prev_scatter):
            off = ebase + w * CH
            pltpu.make_async_copy(src_hbm.at[pl.ds(off, CH)], isrc[t],
                                  isem[t]).wait()
            pltpu.make_async_copy(dst_hbm.at[pl.ds(off, CH)], idst[u],
                                  isem[t]).wait()
            if may_have_prev_scatter:
                @pl.when(w >= NSLOT)
                def _():
                    scatter_wait(t, (u - NSLOT) % DS)
            pltpu.async_copy(table_hbm.at[isrc[t]], rows[t], gsem[t])

        for t in range(NSLOT):
            idx_start(t, t, t)
        for t in range(NSLOT - 1):
            gather_start(t, t, t, may_have_prev_scatter=False)

        def grp(k, _):
            for u in range(DS):
                w = k * DS + u
                t = u % NSLOT

                @pl.when(w < NWIN)
                def _drain():
                    pltpu.make_async_copy(table_hbm.at[isrc[t]], rows[t],
                                          gsem[t]).wait()
                    pltpu.async_copy(rows[t], acc.at[idst[u]], ssem[t],
                                     add=True)
                    if with_counts:
                        pltpu.async_copy(ones_v, cacc.at[idst[u]], ssem[t],
                                         add=True)

                @pl.when(w + NSLOT < NWIN)
                def _prefetch():
                    idx_start(t, (u + NSLOT) % DS, w + NSLOT)

                v = w + NSLOT - 1
                uv = (u + NSLOT - 1) % DS
                tv = uv % NSLOT

                @pl.when(v < NWIN)
                def _gather():
                    gather_start(tv, uv, v, may_have_prev_scatter=True)
            return ()

        lax.fori_loop(0, pl.cdiv(NWIN, DS), grp, ())

        for w in range(NWIN - NSLOT, NWIN):
            scatter_wait(w % NSLOT, w % DS)

        plsc.subcore_barrier()
        pltpu.sync_copy(acc.at[pl.ds(r0, rows_per_sub)],
                        out_hbm.at[c, pl.ds(r0, rows_per_sub)])
        if with_counts:
            pltpu.sync_copy(cacc.at[pl.ds(r0, rows_per_sub)], cbuf)
            pltpu.sync_copy(cbuf,
                            cnt_hbm.at[pl.ds(c * n_dst_pad + r0, rows_per_sub)])

    return pl.kernel(body, out_type=tuple(out_type), mesh=mesh,
                     scratch_types=tuple(scratch))


def _seg_sum(src_idx, dst_idx, table, n_dst_pad, with_counts):
    _, epw = _variant(n_dst_pad)
    e_pad = NW * epw
    if e_pad > E:
        n_dst = {P_PAD: N_P, D_PAD: N_D, SE_PAD: N_SE}[n_dst_pad]
        pad_src = jnp.arange(e_pad - E, dtype=jnp.int32) % table.shape[0]
        pad_dst = (n_dst
                   + jnp.arange(e_pad - E, dtype=jnp.int32) % (n_dst_pad - n_dst))
        src_idx = jnp.concatenate([src_idx, pad_src])
        dst_idx = jnp.concatenate([dst_idx, pad_dst])
    zrow = jnp.zeros((n_dst_pad, HID), jnp.float32)
    k = _make_seg_sum(n_dst_pad, with_counts)
    res = k(src_idx, dst_idx, table, zrow)
    if with_counts:
        sums, cnt_flat = res
        cnt = jnp.broadcast_to(
            cnt_flat.reshape(NC, n_dst_pad, 1), (NC, n_dst_pad, CW))
        return sums, cnt
    return res


def _dotT(a, w):
    return lax.dot_general(a, w, (((1,), (1,)), ((), ())),
                           preferred_element_type=jnp.float32)


def _proj(x, w, b, act, bn):
    n = x.shape[0]

    def body(x_ref, w_ref, b_ref, o_ref):
        h = _dotT(x_ref[...], w_ref[...]) + b_ref[...]
        o_ref[...] = jnp.tanh(h) if act else h

    return pl.pallas_call(
        body,
        grid=(n // bn,),
        in_specs=[pl.BlockSpec((bn, IN_DIM), lambda i: (i, 0)),
                  pl.BlockSpec((HID, IN_DIM), lambda i: (0, 0)),
                  pl.BlockSpec((1, HID), lambda i: (0, 0))],
        out_specs=pl.BlockSpec((bn, HID), lambda i: (i, 0)),
        out_shape=jax.ShapeDtypeStruct((n, HID), jnp.float32),
    )(x, w, b.reshape(1, HID))


def _combine(convs, h_dst, wr_sum, b_sum, bn):
    n = h_dst.shape[0]
    nj = len(convs)

    def body(*refs):
        sums = refs[0:nj]
        cnts = refs[nj:2 * nj]
        wls = refs[2 * nj:3 * nj]
        h_ref, wr_ref, b_ref, o_ref = refs[3 * nj:]
        out = _dotT(h_ref[...], wr_ref[...]) + b_ref[...]
        for j in range(nj):
            ssum = sums[j][0] + sums[j][1]
            cnt = cnts[j][0, :, 0:1] + cnts[j][1, :, 0:1]
            agg = ssum / jnp.maximum(cnt, 1.0)
            out = out + _dotT(agg, wls[j][...])
        o_ref[...] = out

    in_specs = (
        [pl.BlockSpec((2, bn, HID), lambda i: (0, i, 0)) for _ in range(nj)]
        + [pl.BlockSpec((2, bn, CW), lambda i: (0, i, 0)) for _ in range(nj)]
        + [pl.BlockSpec((HID, HID), lambda i: (0, 0)) for _ in range(nj)]
        + [pl.BlockSpec((bn, HID), lambda i: (i, 0)),
           pl.BlockSpec((HID, HID), lambda i: (0, 0)),
           pl.BlockSpec((1, HID), lambda i: (0, 0))]
    )
    args = ([c[0] for c in convs] + [c[1] for c in convs]
            + [c[2] for c in convs] + [h_dst, wr_sum, b_sum.reshape(1, HID)])
    return pl.pallas_call(
        body,
        grid=(n // bn,),
        in_specs=in_specs,
        out_specs=pl.BlockSpec((bn, HID), lambda i: (i, 0)),
        out_shape=jax.ShapeDtypeStruct((n, HID), jnp.float32),
    )(*args)


def _readout(h, assis, w, b, n_rows, bn=512):
    n_out = w.shape[0]

    def body(h_ref, a_ref, w_ref, b_ref, o_ref):
        o_ref[...] = _dotT(h_ref[...] + a_ref[...], w_ref[...]) + b_ref[...]

    return pl.pallas_call(
        body,
        grid=(pl.cdiv(n_rows, bn),),
        in_specs=[pl.BlockSpec((bn, HID), lambda i: (i, 0)),
                  pl.BlockSpec((bn, HID), lambda i: (i, 0)),
                  pl.BlockSpec((n_out, HID), lambda i: (0, 0)),
                  pl.BlockSpec((1, n_out), lambda i: (0, 0))],
        out_specs=pl.BlockSpec((bn, n_out), lambda i: (i, 0)),
        out_shape=jax.ShapeDtypeStruct((n_rows, n_out), jnp.float32),
    )(h, assis, w, b.reshape(1, n_out))


def kernel(x_patient, x_drug, x_SE, ei_p_drug, ei_rev_p_drug, ei_p_se,
           ei_rev_p_se, params):
    p = params
    xse = jnp.pad(x_SE, ((0, SE_PAD - N_SE), (0, 0)))

    h_p = _proj(x_patient, p["W_in"], p["b_in"], act=True, bn=400)
    h_d = _proj(x_drug, p["W_in"], p["b_in"], act=True, bn=600)
    h_se = _proj(xse, p["W_se"], p["b_se"], act=True, bn=512)
    assis = _proj(x_patient, p["W_cl"], p["b_cl"], act=False, bn=400)

    edges = {
        "pd": (ei_p_drug[0], ei_p_drug[1], D_PAD),
        "pse": (ei_p_se[0], ei_p_se[1], SE_PAD),
        "rpd": (ei_rev_p_drug[0], ei_rev_p_drug[1], P_PAD),
        "rpse": (ei_rev_p_se[0], ei_rev_p_se[1], P_PAD),
    }
    tables = {"pd": "p", "pse": "p", "rpd": "d", "rpse": "se"}

    cnts = {}
    for l in range(2):
        sp = p["sage"][l]
        h = {"p": h_p, "d": h_d, "se": h_se}
        sums = {}
        for et, (src, dst, npad) in edges.items():
            res = _seg_sum(src, dst, h[tables[et]], npad, with_counts=(l == 0))
            if l == 0:
                sums[et], cnts[et] = res
            else:
                (sums[et],) = res

        h_d = _combine([(sums["pd"], cnts["pd"], sp["pd"]["Wl"])],
                       h["d"], sp["pd"]["Wr"], sp["pd"]["bl"], bn=600)
        h_se = _combine([(sums["pse"], cnts["pse"], sp["pse"]["Wl"])],
                        h["se"], sp["pse"]["Wr"], sp["pse"]["bl"], bn=512)
        h_p = _combine([(sums["rpd"], cnts["rpd"], sp["rpd"]["Wl"]),
                        (sums["rpse"], cnts["rpse"], sp["rpse"]["Wl"])],
                       h["p"], sp["rpd"]["Wr"] + sp["rpse"]["Wr"],
                       sp["rpd"]["bl"] + sp["rpse"]["bl"], bn=400)

    return _readout(h_p, assis, p["W_ro"], p["b_ro"], n_rows=N_P - 1)


# speedup vs baseline: 1.1374x; 1.0186x over previous
import functools

import jax
import jax.numpy as jnp
from jax import lax
from jax.experimental import pallas as pl
from jax.experimental.pallas import tpu as pltpu
from jax.experimental.pallas import tpu_sc as plsc

N_P, N_D, N_SE = 10000, 3000, 994
E = 320000
IN_DIM, HID, OUT = 512, 128, 994

NC, NS = 2, 16
NW = NC * NS

P_PAD, D_PAD, SE_PAD = 10240, 3072, 1024
CW = 8


def _variant(n_dst_pad):
    ch = 80 if n_dst_pad >= P_PAD else 128
    epw = ch * ((E + NW * ch - 1) // (NW * ch))
    return ch, epw


@functools.lru_cache(maxsize=None)
def _make_seg_sum(n_dst_pad: int, with_counts: bool):
    NSLOT = 4
    CH, EPW = _variant(n_dst_pad)
    NWIN = EPW // CH
    rows_per_sub = n_dst_pad // NS
    mesh = plsc.VectorSubcoreMesh(core_axis_name="c", subcore_axis_name="s",
                                  num_cores=NC, num_subcores=NS)

    out_type = [jax.ShapeDtypeStruct((NC, n_dst_pad, HID), jnp.float32)]
    if with_counts:
        out_type.append(jax.ShapeDtypeStruct((NC * n_dst_pad,), jnp.float32))

    scratch = (
        [pltpu.VMEM((CH,), jnp.int32) for _ in range(NSLOT)]
        + [pltpu.VMEM((CH,), jnp.int32) for _ in range(NSLOT)]
        + [pltpu.VMEM((CH, HID), jnp.float32) for _ in range(NSLOT)]
        + [pltpu.SemaphoreType.DMA for _ in range(NSLOT)]
        + [pltpu.SemaphoreType.DMA for _ in range(NSLOT)]
        + [pltpu.VMEM_SHARED((n_dst_pad, HID), jnp.float32)]
    )
    if with_counts:
        scratch += [pltpu.VMEM((CH,), jnp.float32),
                    pltpu.VMEM((rows_per_sub,), jnp.float32),
                    pltpu.VMEM_SHARED((n_dst_pad,), jnp.float32)]

    def body(src_hbm, dst_hbm, table_hbm, zrow_hbm, *out_and_scratch):
        if with_counts:
            out_hbm, cnt_hbm = out_and_scratch[:2]
            rest = list(out_and_scratch[2:])
        else:
            out_hbm = out_and_scratch[0]
            rest = list(out_and_scratch[1:])
        isrc = [rest.pop(0) for _ in range(NSLOT)]
        idst = [rest.pop(0) for _ in range(NSLOT)]
        rows = [rest.pop(0) for _ in range(NSLOT)]
        gsem = [rest.pop(0) for _ in range(NSLOT)]
        isem = [rest.pop(0) for _ in range(NSLOT)]
        acc = rest.pop(0)
        if with_counts:
            ones_v = rest.pop(0)
            cbuf = rest.pop(0)
            cacc = rest.pop(0)

        c = lax.axis_index("c")
        s = lax.axis_index("s")
        wid = c * NS + s
        ebase = wid * EPW
        r0 = s * rows_per_sub

        pltpu.sync_copy(zrow_hbm.at[pl.ds(r0, rows_per_sub)],
                        acc.at[pl.ds(r0, rows_per_sub)])
        if with_counts:
            for i in range(rows_per_sub // 16):
                cbuf[pl.ds(i * 16, 16)] = jnp.zeros((16,), jnp.float32)
            for i in range(CH // 16):
                ones_v[pl.ds(i * 16, 16)] = jnp.ones((16,), jnp.float32)
            pltpu.sync_copy(cbuf, cacc.at[pl.ds(r0, rows_per_sub)])
        plsc.subcore_barrier()

        def idx_start(t, w):
            off = ebase + w * CH
            pltpu.async_copy(src_hbm.at[pl.ds(off, CH)], isrc[t], isem[t])
            pltpu.async_copy(dst_hbm.at[pl.ds(off, CH)], idst[t], isem[t])

        def gather_start(t, w):
            off = ebase + w * CH
            pltpu.make_async_copy(src_hbm.at[pl.ds(off, CH)], isrc[t],
                                  isem[t]).wait()
            pltpu.make_async_copy(dst_hbm.at[pl.ds(off, CH)], idst[t],
                                  isem[t]).wait()
            pltpu.async_copy(table_hbm.at[isrc[t]], rows[t], gsem[t])

        def drain(t):
            pltpu.make_async_copy(table_hbm.at[isrc[t]], rows[t],
                                  gsem[t]).wait()
            pltpu.sync_copy(rows[t], acc.at[idst[t]], add=True)
            if with_counts:
                pltpu.sync_copy(ones_v, cacc.at[idst[t]], add=True)

        for t in range(NSLOT):
            idx_start(t, t)
        for t in range(NSLOT - 1):
            gather_start(t, t)

        G = (NWIN - NSLOT) // NSLOT

        def grp(k, _):
            for t in range(NSLOT):
                w = k * NSLOT + t
                drain(t)
                idx_start(t, w + NSLOT)
                gather_start((t + NSLOT - 1) % NSLOT, w + NSLOT - 1)
            return ()

        lax.fori_loop(0, G, grp, ())

        for w in range(G * NSLOT, NWIN):
            t = w % NSLOT
            drain(t)
            if w + NSLOT < NWIN:
                idx_start(t, w + NSLOT)
            if w + NSLOT - 1 < NWIN:
                gather_start((t + NSLOT - 1) % NSLOT, w + NSLOT - 1)

        plsc.subcore_barrier()
        pltpu.sync_copy(acc.at[pl.ds(r0, rows_per_sub)],
                        out_hbm.at[c, pl.ds(r0, rows_per_sub)])
        if with_counts:
            pltpu.sync_copy(cacc.at[pl.ds(r0, rows_per_sub)], cbuf)
            pltpu.sync_copy(cbuf,
                            cnt_hbm.at[pl.ds(c * n_dst_pad + r0, rows_per_sub)])

    return pl.kernel(body, out_type=tuple(out_type), mesh=mesh,
                     scratch_types=tuple(scratch))


def _seg_sum(src_idx, dst_idx, table, n_dst_pad, with_counts):
    _, epw = _variant(n_dst_pad)
    e_pad = NW * epw
    if e_pad > E:
        n_dst = {P_PAD: N_P, D_PAD: N_D, SE_PAD: N_SE}[n_dst_pad]
        pad_src = jnp.arange(e_pad - E, dtype=jnp.int32) % table.shape[0]
        pad_dst = (n_dst
                   + jnp.arange(e_pad - E, dtype=jnp.int32) % (n_dst_pad - n_dst))
        src_idx = jnp.concatenate([src_idx, pad_src])
        dst_idx = jnp.concatenate([dst_idx, pad_dst])
    zrow = jnp.zeros((n_dst_pad, HID), jnp.float32)
    k = _make_seg_sum(n_dst_pad, with_counts)
    res = k(src_idx, dst_idx, table, zrow)
    if with_counts:
        sums, cnt_flat = res
        cnt = jnp.broadcast_to(
            cnt_flat.reshape(NC, n_dst_pad, 1), (NC, n_dst_pad, CW))
        return sums, cnt
    return res


def _dotT(a, w):
    return lax.dot_general(a, w, (((1,), (1,)), ((), ())),
                           preferred_element_type=jnp.float32)


def _proj(x, w, b, act, bn):
    n = x.shape[0]

    def body(x_ref, w_ref, b_ref, o_ref):
        h = _dotT(x_ref[...], w_ref[...]) + b_ref[...]
        o_ref[...] = jnp.tanh(h) if act else h

    return pl.pallas_call(
        body,
        grid=(n // bn,),
        in_specs=[pl.BlockSpec((bn, IN_DIM), lambda i: (i, 0)),
                  pl.BlockSpec((HID, IN_DIM), lambda i: (0, 0)),
                  pl.BlockSpec((1, HID), lambda i: (0, 0))],
        out_specs=pl.BlockSpec((bn, HID), lambda i: (i, 0)),
        out_shape=jax.ShapeDtypeStruct((n, HID), jnp.float32),
    )(x, w, b.reshape(1, HID))


def _combine(convs, h_dst, wr_sum, b_sum, bn):
    n = h_dst.shape[0]
    nj = len(convs)

    def body(*refs):
        sums = refs[0:nj]
        cnts = refs[nj:2 * nj]
        wls = refs[2 * nj:3 * nj]
        h_ref, wr_ref, b_ref, o_ref = refs[3 * nj:]
        out = _dotT(h_ref[...], wr_ref[...]) + b_ref[...]
        for j in range(nj):
            ssum = sums[j][0] + sums[j][1]
            cnt = cnts[j][0, :, 0:1] + cnts[j][1, :, 0:1]
            agg = ssum / jnp.maximum(cnt, 1.0)
            out = out + _dotT(agg, wls[j][...])
        o_ref[...] = out

    in_specs = (
        [pl.BlockSpec((2, bn, HID), lambda i: (0, i, 0)) for _ in range(nj)]
        + [pl.BlockSpec((2, bn, CW), lambda i: (0, i, 0)) for _ in range(nj)]
        + [pl.BlockSpec((HID, HID), lambda i: (0, 0)) for _ in range(nj)]
        + [pl.BlockSpec((bn, HID), lambda i: (i, 0)),
           pl.BlockSpec((HID, HID), lambda i: (0, 0)),
           pl.BlockSpec((1, HID), lambda i: (0, 0))]
    )
    args = ([c[0] for c in convs] + [c[1] for c in convs]
            + [c[2] for c in convs] + [h_dst, wr_sum, b_sum.reshape(1, HID)])
    return pl.pallas_call(
        body,
        grid=(n // bn,),
        in_specs=in_specs,
        out_specs=pl.BlockSpec((bn, HID), lambda i: (i, 0)),
        out_shape=jax.ShapeDtypeStruct((n, HID), jnp.float32),
    )(*args)


def _readout(h, assis, w, b, n_rows, bn=512):
    n_out = w.shape[0]

    def body(h_ref, a_ref, w_ref, b_ref, o_ref):
        o_ref[...] = _dotT(h_ref[...] + a_ref[...], w_ref[...]) + b_ref[...]

    return pl.pallas_call(
        body,
        grid=(pl.cdiv(n_rows, bn),),
        in_specs=[pl.BlockSpec((bn, HID), lambda i: (i, 0)),
                  pl.BlockSpec((bn, HID), lambda i: (i, 0)),
                  pl.BlockSpec((n_out, HID), lambda i: (0, 0)),
                  pl.BlockSpec((1, n_out), lambda i: (0, 0))],
        out_specs=pl.BlockSpec((bn, n_out), lambda i: (i, 0)),
        out_shape=jax.ShapeDtypeStruct((n_rows, n_out), jnp.float32),
    )(h, assis, w, b.reshape(1, n_out))


def kernel(x_patient, x_drug, x_SE, ei_p_drug, ei_rev_p_drug, ei_p_se,
           ei_rev_p_se, params):
    p = params
    xse = jnp.pad(x_SE, ((0, SE_PAD - N_SE), (0, 0)))

    h_p = _proj(x_patient, p["W_in"], p["b_in"], act=True, bn=400)
    h_d = _proj(x_drug, p["W_in"], p["b_in"], act=True, bn=600)
    h_se = _proj(xse, p["W_se"], p["b_se"], act=True, bn=512)
    assis = _proj(x_patient, p["W_cl"], p["b_cl"], act=False, bn=400)

    edges = {
        "pd": (ei_p_drug[0], ei_p_drug[1], D_PAD),
        "pse": (ei_p_se[0], ei_p_se[1], SE_PAD),
        "rpd": (ei_rev_p_drug[0], ei_rev_p_drug[1], P_PAD),
        "rpse": (ei_rev_p_se[0], ei_rev_p_se[1], P_PAD),
    }
    tables = {"pd": "p", "pse": "p", "rpd": "d", "rpse": "se"}

    cnts = {}
    for l in range(2):
        sp = p["sage"][l]
        h = {"p": h_p, "d": h_d, "se": h_se}
        sums = {}
        for et, (src, dst, npad) in edges.items():
            res = _seg_sum(src, dst, h[tables[et]], npad, with_counts=(l == 0))
            if l == 0:
                sums[et], cnts[et] = res
            else:
                (sums[et],) = res

        h_d = _combine([(sums["pd"], cnts["pd"], sp["pd"]["Wl"])],
                       h["d"], sp["pd"]["Wr"], sp["pd"]["bl"], bn=600)
        h_se = _combine([(sums["pse"], cnts["pse"], sp["pse"]["Wl"])],
                        h["se"], sp["pse"]["Wr"], sp["pse"]["bl"], bn=512)
        h_p = _combine([(sums["rpd"], cnts["rpd"], sp["rpd"]["Wl"]),
                        (sums["rpse"], cnts["rpse"], sp["rpse"]["Wl"])],
                       h["p"], sp["rpd"]["Wr"] + sp["rpse"]["Wr"],
                       sp["rpd"]["bl"] + sp["rpse"]["bl"], bn=400)

    return _readout(h_p, assis, p["W_ro"], p["b_ro"], n_rows=N_P - 1)
